# Initial kernel scaffold; baseline (speedup 1.0000x reference)
import jax
import jax.numpy as jnp
from jax import lax
from jax.experimental import pallas as pl
from jax.experimental.pallas import tpu as pltpu
from jax.experimental.pallas import tpu_sc as plsc

_NUM_LEVELS = 16
_TABLE_SIZE = 524288
_FEAT = 2
_N = 262144
_NC, _NS = 2, 16
_NW = _NC * _NS
_S = _N // _NW
_C = 2048
_NCHUNK = _S // _C
_G = _C // 16
_ROWS = (_C * _NUM_LEVELS) // 128

_P1 = -1640531535
_P2 = 805459861
_MASK = _TABLE_SIZE - 1


def _sc_body(pos_ref, scal_ref, table_ref, out_ref, posbuf, scalbuf, idxbuf,
             rowsbuf, sem):
    cid = lax.axis_index("c")
    sid = lax.axis_index("s")
    wid = sid * _NC + cid
    base = wid * _S
    pltpu.sync_copy(scal_ref, scalbuf)
    iota = lax.iota(jnp.int32, 16)
    lane_col = (iota & 7) * _NUM_LEVELS
    lane_hi = (iota >= 8).astype(jnp.int32)

    def chunk_body(k, carry):
        cb = base + k * _C
        pltpu.sync_copy(pos_ref.at[pl.ds(3 * cb, 3 * _C)], posbuf)

        def group_body(g, c2):
            off = 48 * g + 3 * iota
            xs = plsc.load_gather(posbuf, [off])
            ys = plsc.load_gather(posbuf, [off + 1])
            zs = plsc.load_gather(posbuf, [off + 2])
            row = 2 * g + lane_hi
            for l in range(_NUM_LEVELS):
                s = scalbuf[l]
                sx = xs * s
                sy = ys * s
                sz = zs * s
                cx = sx.astype(jnp.int32)
                cy = sy.astype(jnp.int32)
                cz = sz.astype(jnp.int32)
                cx = cx + (sx > cx.astype(jnp.float32)).astype(jnp.int32)
                cy = cy + (sy > cy.astype(jnp.float32)).astype(jnp.int32)
                cz = cz + (sz > cz.astype(jnp.float32)).astype(jnp.int32)
                h = cx ^ (cy * jnp.int32(_P1)) ^ (cz * jnp.int32(_P2))
                h = (h & jnp.int32(_MASK)) + jnp.int32(l * _TABLE_SIZE)
                plsc.store_scatter(idxbuf, [row, lane_col + l], h)
            return c2

        lax.fori_loop(0, _G, group_body, 0)

        def fire(j, c2):
            pltpu.make_async_copy(table_ref.at[idxbuf.at[j]],
                                  rowsbuf.at[pl.ds(j * 128, 128), :],
                                  sem).start()
            return c2

        lax.fori_loop(0, _ROWS, fire, 0)

        pltpu.make_async_copy(table_ref.at[pl.ds(0, _C * _NUM_LEVELS), :],
                              rowsbuf, sem).wait()

        pltpu.sync_copy(rowsbuf,
                        out_ref.at[pl.ds(cb * _NUM_LEVELS, _C * _NUM_LEVELS), :])
        return carry

    lax.fori_loop(0, _NCHUNK, chunk_body, 0)


def _sc_gather(pos_flat, scal_b, hash_table):
    mesh = plsc.VectorSubcoreMesh(core_axis_name="c", subcore_axis_name="s",
                                  num_cores=_NC, num_subcores=_NS)
    return pl.kernel(
        _sc_body,
        out_type=jax.ShapeDtypeStruct((_N * _NUM_LEVELS, _FEAT), jnp.float32),
        mesh=mesh,
        scratch_types=[
            pltpu.VMEM((3 * _C,), jnp.float32),
            pltpu.VMEM((_NUM_LEVELS, 16), jnp.float32),
            pltpu.VMEM((_ROWS, 128), jnp.int32),
            pltpu.VMEM((_C * _NUM_LEVELS, _FEAT), jnp.float32),
            pltpu.SemaphoreType.DMA,
        ],
    )(pos_flat, scal_b, hash_table)


_B = 2048


def _mlp_body(enc_ref, dir_ref, w1, b1, w2, b2, w3, b3, w4, b4, w5, b5,
              dens_ref, col_ref):
    x = enc_ref[...]
    h = jnp.maximum(jnp.dot(x, w1[...], preferred_element_type=jnp.float32)
                    + b1[...], 0.0)
    dens = jnp.dot(h, w2[...], preferred_element_type=jnp.float32) + b2[...]
    dens_ref[...] = dens

    d = dir_ref[...]
    dx = d[:, 0:1]
    dy = d[:, 1:2]
    dz = d[:, 2:3]
    xx = dx * dx
    yy = dy * dy
    zz = dz * dz
    comps = [
        0.28209479177387814 * jnp.ones_like(dx),
        0.4886025119029199 * dy,
        0.4886025119029199 * dz,
        0.4886025119029199 * dx,
        1.0925484305920792 * dx * dy,
        1.0925484305920792 * dy * dz,
        0.9461746957575601 * zz - 0.31539156525252,
        1.0925484305920792 * dx * dz,
        0.5462742152960396 * (xx - yy),
        0.5900435899266435 * dy * (3 * xx - yy),
        2.890611442640554 * dx * dy * dz,
        0.4570457994644658 * dy * (5 * zz - 1),
        0.3731763325901154 * dz * (5 * zz - 3),
        0.4570457994644658 * dx * (5 * zz - 1),
        1.445305721320277 * dz * (xx - yy),
        0.5900435899266435 * dx * (xx - 3 * yy),
    ]
    sh = jnp.concatenate(comps, axis=1)
    xc = jnp.concatenate([dens, sh], axis=1)
    xc = jnp.maximum(jnp.dot(xc, w3[...], preferred_element_type=jnp.float32)
                     + b3[...], 0.0)
    xc = jnp.maximum(jnp.dot(xc, w4[...], preferred_element_type=jnp.float32)
                     + b4[...], 0.0)
    col_ref[...] = jax.nn.sigmoid(
        jnp.dot(xc, w5[...], preferred_element_type=jnp.float32) + b5[...])


def _mlp(enc, direction, W1, b1, W2, b2, W3, b3, W4, b4, W5, b5):
    grid = (_N // _B,)
    full = lambda shape: pl.BlockSpec(shape, lambda i: (0, 0))
    return pl.pallas_call(
        _mlp_body,
        grid=grid,
        in_specs=[
            pl.BlockSpec((_B, _NUM_LEVELS * _FEAT), lambda i: (i, 0)),
            pl.BlockSpec((_B, 3), lambda i: (i, 0)),
            full(W1.shape), full((1, 64)),
            full(W2.shape), full((1, 16)),
            full(W3.shape), full((1, 64)),
            full(W4.shape), full((1, 64)),
            full(W5.shape), full((1, 3)),
        ],
        out_specs=[
            pl.BlockSpec((_B, 16), lambda i: (i, 0)),
            pl.BlockSpec((_B, 3), lambda i: (i, 0)),
        ],
        out_shape=[
            jax.ShapeDtypeStruct((_N, 16), jnp.float32),
            jax.ShapeDtypeStruct((_N, 3), jnp.float32),
        ],
    )(enc, direction, W1, b1.reshape(1, -1), W2, b2.reshape(1, -1),
      W3, b3.reshape(1, -1), W4, b4.reshape(1, -1), W5, b5.reshape(1, -1))


def kernel(position, direction, hash_table, W1, b1, W2, b2, W3, b3, W4, b4,
           W5, b5):
    levels = jnp.arange(_NUM_LEVELS)
    growth = jnp.exp((jnp.log(1024.0) - jnp.log(16.0)) / (_NUM_LEVELS - 1))
    scal = jnp.floor(16 * growth ** levels)
    scal_b = jnp.tile(scal.astype(jnp.float32)[:, None], (1, 16))
    pos_flat = position.reshape(-1)
    enc_rows = _sc_gather(pos_flat, scal_b, hash_table)
    enc = enc_rows.reshape(_N, _NUM_LEVELS * _FEAT)
    density, color = _mlp(enc, direction, W1, b1, W2, b2, W3, b3, W4, b4,
                          W5, b5)
    return density, color


# speedup vs baseline: 5.6963x; 5.6963x over previous
import jax
import jax.numpy as jnp
from jax import lax
from jax.experimental import pallas as pl
from jax.experimental.pallas import tpu as pltpu
from jax.experimental.pallas import tpu_sc as plsc

_NUM_LEVELS = 16
_TABLE_SIZE = 524288
_FEAT = 2
_N = 262144
_NC, _NS = 2, 16
_NW = _NC * _NS
_S = _N // _NW
_C = 1024
_NCHUNK = _S // _C
_G = _C // 16
_ROWS = (_C * _NUM_LEVELS * _FEAT) // 128

_P1 = -1640531535
_P2 = 805459861
_MASK = _TABLE_SIZE - 1


def _sc_body(pos_ref, scal_ref, table_ref, out_ref, posbuf, scalbuf, idxbuf,
             rowsbuf, sem):
    cid = lax.axis_index("c")
    sid = lax.axis_index("s")
    wid = sid * _NC + cid
    base = wid * _S
    pltpu.sync_copy(scal_ref, scalbuf)

    def chunk_body(k, carry):
        cb = base + k * _C
        pltpu.sync_copy(pos_ref.at[pl.ds(3 * cb, 3 * _C)], posbuf)

        def group_body(g, c2):
            iota = lax.iota(jnp.int32, 16)
            lane_col = (iota & 3) * 32
            row = jnp.broadcast_to(4 * g, (16,)) + lax.shift_right_logical(iota, 2)
            off = jnp.broadcast_to(48 * g, (16,)) + 3 * iota
            xs = plsc.load_gather(posbuf, [off])
            ys = plsc.load_gather(posbuf, [off + 1])
            zs = plsc.load_gather(posbuf, [off + 2])
            for l in range(_NUM_LEVELS):
                s = scalbuf[l]
                sx = xs * s
                sy = ys * s
                sz = zs * s
                cx = sx.astype(jnp.int32)
                cy = sy.astype(jnp.int32)
                cz = sz.astype(jnp.int32)
                cx = cx + (sx > cx.astype(jnp.float32)).astype(jnp.int32)
                cy = cy + (sy > cy.astype(jnp.float32)).astype(jnp.int32)
                cz = cz + (sz > cz.astype(jnp.float32)).astype(jnp.int32)
                h = cx ^ (cy * jnp.int32(_P1)) ^ (cz * jnp.int32(_P2))
                h = (h & jnp.int32(_MASK)) + jnp.int32(l * _TABLE_SIZE)
                h2 = h + h
                plsc.store_scatter(idxbuf, [row, lane_col + 2 * l], h2)
                plsc.store_scatter(idxbuf, [row, lane_col + 2 * l + 1], h2 + 1)
            return c2

        lax.fori_loop(0, _G, group_body, 0)

        def fire(j, c2):
            pltpu.make_async_copy(table_ref.at[idxbuf.at[j]],
                                  rowsbuf.at[pl.ds(j * 128, 128)],
                                  sem).start()
            return c2

        lax.fori_loop(0, _ROWS, fire, 0)

        def drain(j, c2):
            pltpu.make_async_copy(table_ref.at[idxbuf.at[j]],
                                  rowsbuf.at[pl.ds(j * 128, 128)],
                                  sem).wait()
            return c2

        lax.fori_loop(0, _ROWS, drain, 0)

        pltpu.sync_copy(rowsbuf,
                        out_ref.at[pl.ds(cb * _NUM_LEVELS * _FEAT,
                                         _C * _NUM_LEVELS * _FEAT)])
        return carry

    lax.fori_loop(0, _NCHUNK, chunk_body, 0)


def _sc_gather(pos_flat, scal_b, hash_table):
    mesh = plsc.VectorSubcoreMesh(core_axis_name="c", subcore_axis_name="s",
                                  num_cores=_NC, num_subcores=_NS)
    return pl.kernel(
        _sc_body,
        out_type=jax.ShapeDtypeStruct((_N * _NUM_LEVELS * _FEAT,), jnp.float32),
        mesh=mesh,
        scratch_types=[
            pltpu.VMEM((3 * _C,), jnp.float32),
            pltpu.VMEM((_NUM_LEVELS, 16), jnp.float32),
            pltpu.VMEM((_ROWS, 128), jnp.int32),
            pltpu.VMEM((_C * _NUM_LEVELS * _FEAT,), jnp.float32),
            pltpu.SemaphoreType.DMA,
        ],
        compiler_params=pltpu.CompilerParams(use_tc_tiling_on_sc=False,
                                             needs_layout_passes=False),
    )(pos_flat, scal_b, hash_table)


_B = 2048


def _mlp_body(enc_ref, dir_ref, w1, b1, w2, b2, w3, b3, w4, b4, w5, b5,
              dens_ref, col_ref):
    x = enc_ref[...]
    h = jnp.maximum(jnp.dot(x, w1[...], preferred_element_type=jnp.float32)
                    + b1[...], 0.0)
    dens = jnp.dot(h, w2[...], preferred_element_type=jnp.float32) + b2[...]
    dens_ref[...] = dens

    d = dir_ref[...]
    dx = d[:, 0:1]
    dy = d[:, 1:2]
    dz = d[:, 2:3]
    xx = dx * dx
    yy = dy * dy
    zz = dz * dz
    comps = [
        0.28209479177387814 * jnp.ones_like(dx),
        0.4886025119029199 * dy,
        0.4886025119029199 * dz,
        0.4886025119029199 * dx,
        1.0925484305920792 * dx * dy,
        1.0925484305920792 * dy * dz,
        0.9461746957575601 * zz - 0.31539156525252,
        1.0925484305920792 * dx * dz,
        0.5462742152960396 * (xx - yy),
        0.5900435899266435 * dy * (3 * xx - yy),
        2.890611442640554 * dx * dy * dz,
        0.4570457994644658 * dy * (5 * zz - 1),
        0.3731763325901154 * dz * (5 * zz - 3),
        0.4570457994644658 * dx * (5 * zz - 1),
        1.445305721320277 * dz * (xx - yy),
        0.5900435899266435 * dx * (xx - 3 * yy),
    ]
    sh = jnp.concatenate(comps, axis=1)
    xc = jnp.concatenate([dens, sh], axis=1)
    xc = jnp.maximum(jnp.dot(xc, w3[...], preferred_element_type=jnp.float32)
                     + b3[...], 0.0)
    xc = jnp.maximum(jnp.dot(xc, w4[...], preferred_element_type=jnp.float32)
                     + b4[...], 0.0)
    col_ref[...] = jax.nn.sigmoid(
        jnp.dot(xc, w5[...], preferred_element_type=jnp.float32) + b5[...])


def _mlp(enc, direction, W1, b1, W2, b2, W3, b3, W4, b4, W5, b5):
    grid = (_N // _B,)
    full = lambda shape: pl.BlockSpec(shape, lambda i: (0, 0))
    return pl.pallas_call(
        _mlp_body,
        grid=grid,
        in_specs=[
            pl.BlockSpec((_B, _NUM_LEVELS * _FEAT), lambda i: (i, 0)),
            pl.BlockSpec((_B, 3), lambda i: (i, 0)),
            full(W1.shape), full((1, 64)),
            full(W2.shape), full((1, 16)),
            full(W3.shape), full((1, 64)),
            full(W4.shape), full((1, 64)),
            full(W5.shape), full((1, 3)),
        ],
        out_specs=[
            pl.BlockSpec((_B, 16), lambda i: (i, 0)),
            pl.BlockSpec((_B, 3), lambda i: (i, 0)),
        ],
        out_shape=[
            jax.ShapeDtypeStruct((_N, 16), jnp.float32),
            jax.ShapeDtypeStruct((_N, 3), jnp.float32),
        ],
    )(enc, direction, W1, b1.reshape(1, -1), W2, b2.reshape(1, -1),
      W3, b3.reshape(1, -1), W4, b4.reshape(1, -1), W5, b5.reshape(1, -1))


def kernel(position, direction, hash_table, W1, b1, W2, b2, W3, b3, W4, b4,
           W5, b5):
    levels = jnp.arange(_NUM_LEVELS)
    growth = jnp.exp((jnp.log(1024.0) - jnp.log(16.0)) / (_NUM_LEVELS - 1))
    scal = jnp.floor(16 * growth ** levels)
    scal_b = jnp.tile(scal.astype(jnp.float32)[:, None], (1, 16))
    pos_flat = position.reshape(-1)
    enc_rows = _sc_gather(pos_flat, scal_b, hash_table.reshape(-1))
    enc = enc_rows.reshape(_N, _NUM_LEVELS * _FEAT)
    density, color = _mlp(enc, direction, W1, b1, W2, b2, W3, b3, W4, b4,
                          W5, b5)
    return density, color


# speedup vs baseline: 39.9049x; 7.0054x over previous
import jax
import jax.numpy as jnp
from jax import lax
from jax.experimental import pallas as pl
from jax.experimental.pallas import tpu as pltpu
from jax.experimental.pallas import tpu_sc as plsc

_NUM_LEVELS = 16
_TABLE_SIZE = 524288
_FEAT = 2
_N = 262144
_NC, _NS = 2, 16
_NW = _NC * _NS
_S = _N // _NW
_C = 1024
_NCHUNK = _S // _C
_G = _C // 16
_ROWS = (_C * _NUM_LEVELS * _FEAT) // 128

_P1 = -1640531535
_P2 = 805459861
_MASK = _TABLE_SIZE - 1


def _sc_body(pos_ref, scal_ref, table_ref, out_ref, posbuf, scalbuf, idxbuf,
             rowsbuf, sem):
    cid = lax.axis_index("c")
    sid = lax.axis_index("s")
    wid = sid * _NC + cid
    base = wid * _S
    pltpu.sync_copy(scal_ref, scalbuf)

    def chunk_body(k, carry):
        cb = base + k * _C
        pltpu.sync_copy(pos_ref.at[pl.ds(3 * cb, 3 * _C)], posbuf)

        def group_body(g, c2):
            iota = lax.iota(jnp.int32, 16)
            lane_col = (iota & 3) * 32
            row = jnp.broadcast_to(4 * g, (16,)) + lax.shift_right_logical(iota, 2)
            off = jnp.broadcast_to(48 * g, (16,)) + 3 * iota
            xs = plsc.load_gather(posbuf, [off])
            ys = plsc.load_gather(posbuf, [off + 1])
            zs = plsc.load_gather(posbuf, [off + 2])
            for l in range(_NUM_LEVELS):
                s = scalbuf[l]
                sx = xs * s
                sy = ys * s
                sz = zs * s
                cx = sx.astype(jnp.int32)
                cy = sy.astype(jnp.int32)
                cz = sz.astype(jnp.int32)
                cx = cx + (sx > cx.astype(jnp.float32)).astype(jnp.int32)
                cy = cy + (sy > cy.astype(jnp.float32)).astype(jnp.int32)
                cz = cz + (sz > cz.astype(jnp.float32)).astype(jnp.int32)
                h = cx ^ (cy * jnp.int32(_P1)) ^ (cz * jnp.int32(_P2))
                h = (h & jnp.int32(_MASK)) + jnp.int32(l * _TABLE_SIZE)
                q0 = (lax.shift_right_logical(h, 7) * 256) + (h & 127)
                plsc.store_scatter(idxbuf, [row, lane_col + 2 * l], q0)
                plsc.store_scatter(idxbuf, [row, lane_col + 2 * l + 1],
                                   q0 + 128)
            return c2

        lax.fori_loop(0, _G, group_body, 0)

        def fire(j, c2):
            pltpu.make_async_copy(table_ref.at[idxbuf.at[j]],
                                  rowsbuf.at[pl.ds(j * 128, 128)],
                                  sem).start()
            return c2

        lax.fori_loop(0, _ROWS, fire, 0)

        def drain(j, c2):
            pltpu.make_async_copy(table_ref.at[idxbuf.at[j]],
                                  rowsbuf.at[pl.ds(j * 128, 128)],
                                  sem).wait()
            return c2

        lax.fori_loop(0, _ROWS, drain, 0)

        pltpu.sync_copy(rowsbuf,
                        out_ref.at[pl.ds(cb * _NUM_LEVELS * _FEAT,
                                         _C * _NUM_LEVELS * _FEAT)])
        return carry

    lax.fori_loop(0, _NCHUNK, chunk_body, 0)


def _sc_gather(pos_flat, scal_b, hash_table):
    mesh = plsc.VectorSubcoreMesh(core_axis_name="c", subcore_axis_name="s",
                                  num_cores=_NC, num_subcores=_NS)
    return pl.kernel(
        _sc_body,
        out_type=jax.ShapeDtypeStruct((_N * _NUM_LEVELS * _FEAT,), jnp.float32),
        mesh=mesh,
        scratch_types=[
            pltpu.VMEM((3 * _C,), jnp.float32),
            pltpu.VMEM((_NUM_LEVELS, 16), jnp.float32),
            pltpu.VMEM((_ROWS, 128), jnp.int32),
            pltpu.VMEM((_C * _NUM_LEVELS * _FEAT,), jnp.float32),
            pltpu.SemaphoreType.DMA,
        ],
        compiler_params=pltpu.CompilerParams(use_tc_tiling_on_sc=False,
                                             needs_layout_passes=False),
    )(pos_flat, scal_b, hash_table)


_B = 2048


def _mlp_body(enc_ref, dir_ref, w1, b1, w2, b2, w3, b3, w4, b4, w5, b5,
              dens_ref, col_ref):
    x = enc_ref[...]
    h = jnp.maximum(jnp.dot(x, w1[...], preferred_element_type=jnp.float32)
                    + b1[...], 0.0)
    dens = jnp.dot(h, w2[...], preferred_element_type=jnp.float32) + b2[...]
    dens_ref[...] = dens

    d = dir_ref[...]
    dx = d[:, 0:1]
    dy = d[:, 1:2]
    dz = d[:, 2:3]
    xx = dx * dx
    yy = dy * dy
    zz = dz * dz
    comps = [
        0.28209479177387814 * jnp.ones_like(dx),
        0.4886025119029199 * dy,
        0.4886025119029199 * dz,
        0.4886025119029199 * dx,
        1.0925484305920792 * dx * dy,
        1.0925484305920792 * dy * dz,
        0.9461746957575601 * zz - 0.31539156525252,
        1.0925484305920792 * dx * dz,
        0.5462742152960396 * (xx - yy),
        0.5900435899266435 * dy * (3 * xx - yy),
        2.890611442640554 * dx * dy * dz,
        0.4570457994644658 * dy * (5 * zz - 1),
        0.3731763325901154 * dz * (5 * zz - 3),
        0.4570457994644658 * dx * (5 * zz - 1),
        1.445305721320277 * dz * (xx - yy),
        0.5900435899266435 * dx * (xx - 3 * yy),
    ]
    sh = jnp.concatenate(comps, axis=1)
    xc = jnp.concatenate([dens, sh], axis=1)
    xc = jnp.maximum(jnp.dot(xc, w3[...], preferred_element_type=jnp.float32)
                     + b3[...], 0.0)
    xc = jnp.maximum(jnp.dot(xc, w4[...], preferred_element_type=jnp.float32)
                     + b4[...], 0.0)
    col_ref[...] = jax.nn.sigmoid(
        jnp.dot(xc, w5[...], preferred_element_type=jnp.float32) + b5[...])


def _mlp(enc, direction, W1, b1, W2, b2, W3, b3, W4, b4, W5, b5):
    grid = (_N // _B,)
    full = lambda shape: pl.BlockSpec(shape, lambda i: (0, 0))
    return pl.pallas_call(
        _mlp_body,
        grid=grid,
        in_specs=[
            pl.BlockSpec((_B, _NUM_LEVELS * _FEAT), lambda i: (i, 0)),
            pl.BlockSpec((_B, 3), lambda i: (i, 0)),
            full(W1.shape), full((1, 64)),
            full(W2.shape), full((1, 16)),
            full(W3.shape), full((1, 64)),
            full(W4.shape), full((1, 64)),
            full(W5.shape), full((1, 3)),
        ],
        out_specs=[
            pl.BlockSpec((_B, 16), lambda i: (i, 0)),
            pl.BlockSpec((_B, 3), lambda i: (i, 0)),
        ],
        out_shape=[
            jax.ShapeDtypeStruct((_N, 16), jnp.float32),
            jax.ShapeDtypeStruct((_N, 3), jnp.float32),
        ],
    )(enc, direction, W1, b1.reshape(1, -1), W2, b2.reshape(1, -1),
      W3, b3.reshape(1, -1), W4, b4.reshape(1, -1), W5, b5.reshape(1, -1))


def kernel(position, direction, hash_table, W1, b1, W2, b2, W3, b3, W4, b4,
           W5, b5):
    levels = jnp.arange(_NUM_LEVELS)
    growth = jnp.exp((jnp.log(1024.0) - jnp.log(16.0)) / (_NUM_LEVELS - 1))
    scal = jnp.floor(16 * growth ** levels)
    scal_b = jnp.tile(scal.astype(jnp.float32)[:, None], (1, 16))
    pos_flat = position.reshape(-1)
    ht_flat = hash_table.reshape(_TABLE_SIZE * _NUM_LEVELS // 128, 128,
                                 _FEAT).transpose(0, 2, 1).reshape(-1)
    enc_rows = _sc_gather(pos_flat, scal_b, ht_flat)
    enc = enc_rows.reshape(_N, _NUM_LEVELS * _FEAT)
    density, color = _mlp(enc, direction, W1, b1, W2, b2, W3, b3, W4, b4,
                          W5, b5)
    return density, color


# speedup vs baseline: 68.3513x; 1.7129x over previous
import jax
import jax.numpy as jnp
from jax import lax
from jax.experimental import pallas as pl
from jax.experimental.pallas import tpu as pltpu
from jax.experimental.pallas import tpu_sc as plsc

_NUM_LEVELS = 16
_TABLE_SIZE = 524288
_FEAT = 2
_N = 262144
_NC, _NS = 2, 16
_NW = _NC * _NS
_S = _N // _NW
_C = 512
_NCHUNK = _S // _C
_G = _C // 16
_ROWS = (_C * _NUM_LEVELS * _FEAT) // 128
_FU = 4

_P1 = -1640531535
_P2 = 805459861
_MASK = _TABLE_SIZE - 1


def _sc_body(pos_ref, scal_ref, table_ref, out_ref, posbuf, scalbuf, idxbuf0,
             idxbuf1, rowsbuf0, rowsbuf1, sem0, sem1):
    cid = lax.axis_index("c")
    sid = lax.axis_index("s")
    wid = sid * _NC + cid
    base = wid * _S
    pltpu.sync_copy(scal_ref, scalbuf)

    def compute(k, idxbuf):
        cb = base + k * _C
        pltpu.sync_copy(pos_ref.at[pl.ds(3 * cb, 3 * _C)], posbuf)

        def group_body(g, c2):
            iota = lax.iota(jnp.int32, 16)
            lane_col = (iota & 3) * 32
            row = jnp.broadcast_to(4 * g, (16,)) + lax.shift_right_logical(iota, 2)
            off = jnp.broadcast_to(48 * g, (16,)) + 3 * iota
            xs = plsc.load_gather(posbuf, [off])
            ys = plsc.load_gather(posbuf, [off + 1])
            zs = plsc.load_gather(posbuf, [off + 2])
            for l in range(_NUM_LEVELS):
                s = scalbuf[l]
                sx = xs * s
                sy = ys * s
                sz = zs * s
                cx = sx.astype(jnp.int32)
                cy = sy.astype(jnp.int32)
                cz = sz.astype(jnp.int32)
                cx = cx + (sx > cx.astype(jnp.float32)).astype(jnp.int32)
                cy = cy + (sy > cy.astype(jnp.float32)).astype(jnp.int32)
                cz = cz + (sz > cz.astype(jnp.float32)).astype(jnp.int32)
                h = cx ^ (cy * jnp.int32(_P1)) ^ (cz * jnp.int32(_P2))
                h = (h & jnp.int32(_MASK)) + jnp.int32(l * _TABLE_SIZE)
                q0 = (lax.shift_right_logical(h, 7) * 256) + (h & 127)
                plsc.store_scatter(idxbuf, [row, lane_col + 2 * l], q0)
                plsc.store_scatter(idxbuf, [row, lane_col + 2 * l + 1],
                                   q0 + 128)
            return c2

        lax.fori_loop(0, _G, group_body, 0)

    def fire(idxbuf, rowsbuf, sem):
        def body(j, c2):
            for i in range(_FU):
                jj = j * _FU + i
                pltpu.make_async_copy(table_ref.at[idxbuf.at[jj]],
                                      rowsbuf.at[pl.ds(jj * 128, 128)],
                                      sem).start()
            return c2

        lax.fori_loop(0, _ROWS // _FU, body, 0)

    def drain(idxbuf, rowsbuf, sem):
        def body(j, c2):
            for i in range(_FU):
                jj = j * _FU + i
                pltpu.make_async_copy(table_ref.at[idxbuf.at[jj]],
                                      rowsbuf.at[pl.ds(jj * 128, 128)],
                                      sem).wait()
            return c2

        lax.fori_loop(0, _ROWS // _FU, body, 0)

    def writeback(k, rowsbuf):
        cb = base + k * _C
        pltpu.sync_copy(rowsbuf,
                        out_ref.at[pl.ds(cb * _NUM_LEVELS * _FEAT,
                                         _C * _NUM_LEVELS * _FEAT)])

    compute(0, idxbuf0)
    fire(idxbuf0, rowsbuf0, sem0)

    def pipe(k, carry):
        @pl.when(k % 2 == 1)
        def _():
            compute(k, idxbuf1)
            fire(idxbuf1, rowsbuf1, sem1)
            drain(idxbuf0, rowsbuf0, sem0)
            writeback(k - 1, rowsbuf0)

        @pl.when(k % 2 == 0)
        def _():
            compute(k, idxbuf0)
            fire(idxbuf0, rowsbuf0, sem0)
            drain(idxbuf1, rowsbuf1, sem1)
            writeback(k - 1, rowsbuf1)

        return carry

    lax.fori_loop(1, _NCHUNK, pipe, 0)
    drain(idxbuf1, rowsbuf1, sem1)
    writeback(_NCHUNK - 1, rowsbuf1)


def _sc_gather(pos_flat, scal_b, hash_table):
    mesh = plsc.VectorSubcoreMesh(core_axis_name="c", subcore_axis_name="s",
                                  num_cores=_NC, num_subcores=_NS)
    return pl.kernel(
        _sc_body,
        out_type=jax.ShapeDtypeStruct((_N * _NUM_LEVELS * _FEAT,), jnp.float32),
        mesh=mesh,
        scratch_types=[
            pltpu.VMEM((3 * _C,), jnp.float32),
            pltpu.VMEM((_NUM_LEVELS, 16), jnp.float32),
            pltpu.VMEM((_ROWS, 128), jnp.int32),
            pltpu.VMEM((_ROWS, 128), jnp.int32),
            pltpu.VMEM((_C * _NUM_LEVELS * _FEAT,), jnp.float32),
            pltpu.VMEM((_C * _NUM_LEVELS * _FEAT,), jnp.float32),
            pltpu.SemaphoreType.DMA,
            pltpu.SemaphoreType.DMA,
        ],
        compiler_params=pltpu.CompilerParams(use_tc_tiling_on_sc=False,
                                             needs_layout_passes=False),
    )(pos_flat, scal_b, hash_table)


_BM = 512
_M = _N // 4


def _mlp_body(enc_ref, dir_ref, w1, b1, w2, b2, w3a, w3b, b3, w4, b4, w5, b5,
              dens_ref, col_ref):
    x = enc_ref[...]
    h = jnp.maximum(jnp.dot(x, w1[...], preferred_element_type=jnp.float32)
                    + b1[...], 0.0)
    d128 = jnp.dot(h, w2[...], preferred_element_type=jnp.float32) + b2[...]
    dens_ref[...] = d128.T

    dd = dir_ref[...]
    rows = []
    for u in range(4):
        dx = dd[3 * u:3 * u + 1, :]
        dy = dd[3 * u + 1:3 * u + 2, :]
        dz = dd[3 * u + 2:3 * u + 3, :]
        xx = dx * dx
        yy = dy * dy
        zz = dz * dz
        rows += [
            0.28209479177387814 * jnp.ones_like(dx),
            0.4886025119029199 * dy,
            0.4886025119029199 * dz,
            0.4886025119029199 * dx,
            1.0925484305920792 * dx * dy,
            1.0925484305920792 * dy * dz,
            0.9461746957575601 * zz - 0.31539156525252,
            1.0925484305920792 * dx * dz,
            0.5462742152960396 * (xx - yy),
            0.5900435899266435 * dy * (3 * xx - yy),
            2.890611442640554 * dx * dy * dz,
            0.4570457994644658 * dy * (5 * zz - 1),
            0.3731763325901154 * dz * (5 * zz - 3),
            0.4570457994644658 * dx * (5 * zz - 1),
            1.445305721320277 * dz * (xx - yy),
            0.5900435899266435 * dx * (xx - 3 * yy),
        ]
    c4 = jnp.concatenate(rows, axis=0)
    shp = lax.dot_general(c4, w3b[...], (((0,), (0,)), ((), ())),
                          preferred_element_type=jnp.float32)
    densp = jnp.dot(d128, w3a[...], preferred_element_type=jnp.float32)
    xc = jnp.maximum(densp + shp + b3[...], 0.0)
    xc = jnp.maximum(jnp.dot(xc, w4[...], preferred_element_type=jnp.float32)
                     + b4[...], 0.0)
    col = jax.nn.sigmoid(
        jnp.dot(xc, w5[...], preferred_element_type=jnp.float32) + b5[...])
    col_ref[...] = col.T


def _mlp(enc128, dir4, W1, b1, W2, b2, W3, b3, W4, b4, W5, b5):
    from jax.scipy.linalg import block_diag

    def bd4(w):
        return block_diag(w, w, w, w)

    W1_4 = bd4(W1)
    W2_4 = bd4(W2)
    W3a4 = bd4(W3[:16, :])
    W3b4 = bd4(W3[16:, :])
    W4_4 = bd4(W4)
    W5_4 = bd4(W5)
    b1_4 = jnp.tile(b1, 4)[None, :]
    b2_4 = jnp.tile(b2, 4)[None, :]
    b3_4 = jnp.tile(b3, 4)[None, :]
    b4_4 = jnp.tile(b4, 4)[None, :]
    b5_4 = jnp.tile(b5, 4)[None, :]

    grid = (_M // _BM,)
    full = lambda a: pl.BlockSpec(a.shape, lambda i: (0, 0))
    dens_t, col_t = pl.pallas_call(
        _mlp_body,
        grid=grid,
        in_specs=[
            pl.BlockSpec((_BM, 128), lambda i: (i, 0)),
            pl.BlockSpec((12, _BM), lambda i: (0, i)),
            full(W1_4), full(b1_4),
            full(W2_4), full(b2_4),
            full(W3a4), full(W3b4), full(b3_4),
            full(W4_4), full(b4_4),
            full(W5_4), full(b5_4),
        ],
        out_specs=[
            pl.BlockSpec((64, _BM), lambda i: (0, i)),
            pl.BlockSpec((12, _BM), lambda i: (0, i)),
        ],
        out_shape=[
            jax.ShapeDtypeStruct((64, _M), jnp.float32),
            jax.ShapeDtypeStruct((12, _M), jnp.float32),
        ],
    )(enc128, dir4, W1_4, b1_4, W2_4, b2_4, W3a4, W3b4, b3_4, W4_4, b4_4,
      W5_4, b5_4)
    density = dens_t.reshape(4, 16, _M).transpose(2, 0, 1).reshape(_N, 16)
    color = col_t.reshape(4, 3, _M).transpose(2, 0, 1).reshape(_N, 3)
    return density, color


def kernel(position, direction, hash_table, W1, b1, W2, b2, W3, b3, W4, b4,
           W5, b5):
    levels = jnp.arange(_NUM_LEVELS)
    growth = jnp.exp((jnp.log(1024.0) - jnp.log(16.0)) / (_NUM_LEVELS - 1))
    scal = jnp.floor(16 * growth ** levels)
    scal_b = jnp.tile(scal.astype(jnp.float32)[:, None], (1, 16))
    pos_flat = position.reshape(-1)
    ht_flat = hash_table.reshape(_TABLE_SIZE * _NUM_LEVELS // 128, 128,
                                 _FEAT).transpose(0, 2, 1).reshape(-1)
    enc_rows = _sc_gather(pos_flat, scal_b, ht_flat)
    enc128 = enc_rows.reshape(_M, 128)
    dir4 = direction.reshape(_M, 4, 3).transpose(1, 2, 0).reshape(12, _M)
    density, color = _mlp(enc128, dir4, W1, b1, W2, b2, W3, b3, W4, b4,
                          W5, b5)
    return density, color


# speedup vs baseline: 68.3565x; 1.0001x over previous
import jax
import jax.numpy as jnp
from jax import lax
from jax.experimental import pallas as pl
from jax.experimental.pallas import tpu as pltpu
from jax.experimental.pallas import tpu_sc as plsc

_NUM_LEVELS = 16
_TABLE_SIZE = 524288
_FEAT = 2
_N = 262144
_NC, _NS = 2, 16
_NW = _NC * _NS
_S = _N // _NW
_C = 512
_NCHUNK = _S // _C
_G = _C // 16
_ROWS = (_C * _NUM_LEVELS * _FEAT) // 128
_FU = 4

_P1 = -1640531535
_P2 = 805459861
_MASK = _TABLE_SIZE - 1


def _sc_body(pos_ref, scal_ref, table_ref, out_ref, posbuf, scalbuf, idxbuf0,
             idxbuf1, rowsbuf0, rowsbuf1, sem0, sem1):
    cid = lax.axis_index("c")
    sid = lax.axis_index("s")
    wid = sid * _NC + cid
    base = wid * _S
    pltpu.sync_copy(scal_ref, scalbuf)

    def compute(k, idxbuf):
        cb = base + k * _C
        pltpu.sync_copy(pos_ref.at[pl.ds(3 * cb, 3 * _C)], posbuf)

        def group_body(g, c2):
            iota = lax.iota(jnp.int32, 16)
            lane_col = (iota & 3) * 32
            row = jnp.broadcast_to(4 * g, (16,)) + lax.shift_right_logical(iota, 2)
            off = jnp.broadcast_to(48 * g, (16,)) + 3 * iota
            xs = plsc.load_gather(posbuf, [off])
            ys = plsc.load_gather(posbuf, [off + 1])
            zs = plsc.load_gather(posbuf, [off + 2])
            for l in range(_NUM_LEVELS):
                s = scalbuf[l]
                sx = xs * s
                sy = ys * s
                sz = zs * s
                cx = sx.astype(jnp.int32)
                cy = sy.astype(jnp.int32)
                cz = sz.astype(jnp.int32)
                cx = cx + (sx > cx.astype(jnp.float32)).astype(jnp.int32)
                cy = cy + (sy > cy.astype(jnp.float32)).astype(jnp.int32)
                cz = cz + (sz > cz.astype(jnp.float32)).astype(jnp.int32)
                h = cx ^ (cy * jnp.int32(_P1)) ^ (cz * jnp.int32(_P2))
                h = (h & jnp.int32(_MASK)) + jnp.int32(l * _TABLE_SIZE)
                q0 = (lax.shift_right_logical(h, 7) * 256) + (h & 127)
                plsc.store_scatter(idxbuf, [row, lane_col + 2 * l], q0)
                plsc.store_scatter(idxbuf, [row, lane_col + 2 * l + 1],
                                   q0 + 128)
            return c2

        lax.fori_loop(0, _G, group_body, 0)

    def fire(idxbuf, rowsbuf, sem):
        def body(j, c2):
            for i in range(_FU):
                jj = j * _FU + i
                pltpu.make_async_copy(table_ref.at[idxbuf.at[jj]],
                                      rowsbuf.at[pl.ds(jj * 128, 128)],
                                      sem).start()
            return c2

        lax.fori_loop(0, _ROWS // _FU, body, 0)

    def drain(idxbuf, rowsbuf, sem):
        def body(j, c2):
            for i in range(_FU):
                jj = j * _FU + i
                pltpu.make_async_copy(table_ref.at[idxbuf.at[jj]],
                                      rowsbuf.at[pl.ds(jj * 128, 128)],
                                      sem).wait()
            return c2

        lax.fori_loop(0, _ROWS // _FU, body, 0)

    def writeback(k, rowsbuf):
        cb = base + k * _C
        pltpu.sync_copy(rowsbuf,
                        out_ref.at[pl.ds(cb * _NUM_LEVELS * _FEAT,
                                         _C * _NUM_LEVELS * _FEAT)])

    compute(0, idxbuf0)
    fire(idxbuf0, rowsbuf0, sem0)

    def pipe(k, carry):
        @pl.when(k % 2 == 1)
        def _():
            compute(k, idxbuf1)
            fire(idxbuf1, rowsbuf1, sem1)
            drain(idxbuf0, rowsbuf0, sem0)
            writeback(k - 1, rowsbuf0)

        @pl.when(k % 2 == 0)
        def _():
            compute(k, idxbuf0)
            fire(idxbuf0, rowsbuf0, sem0)
            drain(idxbuf1, rowsbuf1, sem1)
            writeback(k - 1, rowsbuf1)

        return carry

    lax.fori_loop(1, _NCHUNK, pipe, 0)
    drain(idxbuf1, rowsbuf1, sem1)
    writeback(_NCHUNK - 1, rowsbuf1)


def _sc_gather(pos_flat, scal_b, hash_table):
    mesh = plsc.VectorSubcoreMesh(core_axis_name="c", subcore_axis_name="s",
                                  num_cores=_NC, num_subcores=_NS)
    return pl.kernel(
        _sc_body,
        out_type=jax.ShapeDtypeStruct((_N * _NUM_LEVELS * _FEAT,), jnp.float32),
        mesh=mesh,
        scratch_types=[
            pltpu.VMEM((3 * _C,), jnp.float32),
            pltpu.VMEM((_NUM_LEVELS, 16), jnp.float32),
            pltpu.VMEM((_ROWS, 128), jnp.int32),
            pltpu.VMEM((_ROWS, 128), jnp.int32),
            pltpu.VMEM((_C * _NUM_LEVELS * _FEAT,), jnp.float32),
            pltpu.VMEM((_C * _NUM_LEVELS * _FEAT,), jnp.float32),
            pltpu.SemaphoreType.DMA,
            pltpu.SemaphoreType.DMA,
        ],
        compiler_params=pltpu.CompilerParams(use_tc_tiling_on_sc=False,
                                             needs_layout_passes=False),
    )(pos_flat, scal_b, hash_table)


_BM = 512
_M = _N // 4


def _mlp_body(enc_ref, dir_ref, w1, b1, w2, b2, w3a, w3b, b3, w4, b4, w5, b5,
              dens_ref, col_ref):
    x = enc_ref[...]
    h = jnp.maximum(jnp.dot(x, w1[...], preferred_element_type=jnp.float32)
                    + b1[...], 0.0)
    d128 = jnp.dot(h, w2[...], preferred_element_type=jnp.float32) + b2[...]
    dens_ref[...] = d128.T

    dd = dir_ref[...]
    rows = []
    for u in range(4):
        dx = dd[3 * u:3 * u + 1, :]
        dy = dd[3 * u + 1:3 * u + 2, :]
        dz = dd[3 * u + 2:3 * u + 3, :]
        xx = dx * dx
        yy = dy * dy
        zz = dz * dz
        rows += [
            0.28209479177387814 * jnp.ones_like(dx),
            0.4886025119029199 * dy,
            0.4886025119029199 * dz,
            0.4886025119029199 * dx,
            1.0925484305920792 * dx * dy,
            1.0925484305920792 * dy * dz,
            0.9461746957575601 * zz - 0.31539156525252,
            1.0925484305920792 * dx * dz,
            0.5462742152960396 * (xx - yy),
            0.5900435899266435 * dy * (3 * xx - yy),
            2.890611442640554 * dx * dy * dz,
            0.4570457994644658 * dy * (5 * zz - 1),
            0.3731763325901154 * dz * (5 * zz - 3),
            0.4570457994644658 * dx * (5 * zz - 1),
            1.445305721320277 * dz * (xx - yy),
            0.5900435899266435 * dx * (xx - 3 * yy),
        ]
    c4 = jnp.concatenate(rows, axis=0)
    shp = lax.dot_general(c4, w3b[...], (((0,), (0,)), ((), ())),
                          preferred_element_type=jnp.float32)
    densp = jnp.dot(d128, w3a[...], preferred_element_type=jnp.float32)
    xc = jnp.maximum(densp + shp + b3[...], 0.0)
    xc = jnp.maximum(jnp.dot(xc, w4[...], preferred_element_type=jnp.float32)
                     + b4[...], 0.0)
    col = jax.nn.sigmoid(
        jnp.dot(xc, w5[...], preferred_element_type=jnp.float32) + b5[...])
    col_ref[...] = col.T


def _mlp(enc128, dir4, W1, b1, W2, b2, W3, b3, W4, b4, W5, b5):
    from jax.scipy.linalg import block_diag

    def bd4(w):
        return block_diag(w, w, w, w)

    W1_4 = bd4(W1)
    W2_4 = bd4(W2)
    W3a4 = bd4(W3[:16, :])
    W3b4 = bd4(W3[16:, :])
    W4_4 = bd4(W4)
    W5_4 = bd4(W5)
    b1_4 = jnp.tile(b1, 4)[None, :]
    b2_4 = jnp.tile(b2, 4)[None, :]
    b3_4 = jnp.tile(b3, 4)[None, :]
    b4_4 = jnp.tile(b4, 4)[None, :]
    b5_4 = jnp.tile(b5, 4)[None, :]

    grid = (_M // _BM,)
    full = lambda a: pl.BlockSpec(a.shape, lambda i: (0, 0))
    dens_t, col_t = pl.pallas_call(
        _mlp_body,
        grid=grid,
        in_specs=[
            pl.BlockSpec((_BM, 128), lambda i: (i, 0)),
            pl.BlockSpec((12, _BM), lambda i: (0, i)),
            full(W1_4), full(b1_4),
            full(W2_4), full(b2_4),
            full(W3a4), full(W3b4), full(b3_4),
            full(W4_4), full(b4_4),
            full(W5_4), full(b5_4),
        ],
        out_specs=[
            pl.BlockSpec((64, _BM), lambda i: (0, i)),
            pl.BlockSpec((12, _BM), lambda i: (0, i)),
        ],
        out_shape=[
            jax.ShapeDtypeStruct((64, _M), jnp.float32),
            jax.ShapeDtypeStruct((12, _M), jnp.float32),
        ],
    )(enc128, dir4, W1_4, b1_4, W2_4, b2_4, W3a4, W3b4, b3_4, W4_4, b4_4,
      W5_4, b5_4)
    density = dens_t.reshape(4, 16, _M).transpose(1, 2, 0).reshape(16, _N).T
    color = col_t.reshape(4, 3, _M).transpose(1, 2, 0).reshape(3, _N).T
    return density, color


def kernel(position, direction, hash_table, W1, b1, W2, b2, W3, b3, W4, b4,
           W5, b5):
    levels = jnp.arange(_NUM_LEVELS)
    growth = jnp.exp((jnp.log(1024.0) - jnp.log(16.0)) / (_NUM_LEVELS - 1))
    scal = jnp.floor(16 * growth ** levels)
    scal_b = jnp.tile(scal.astype(jnp.float32)[:, None], (1, 16))
    pos_flat = position.reshape(-1)
    ht_flat = hash_table.reshape(_TABLE_SIZE * _NUM_LEVELS // 128, 128,
                                 _FEAT).transpose(0, 2, 1).reshape(-1)
    enc_rows = _sc_gather(pos_flat, scal_b, ht_flat)
    enc128 = enc_rows.reshape(_M, 128)
    dir4 = direction.reshape(_M, 4, 3).transpose(1, 2, 0).reshape(12, _M)
    density, color = _mlp(enc128, dir4, W1, b1, W2, b2, W3, b3, W4, b4,
                          W5, b5)
    return density, color


# speedup vs baseline: 75.4604x; 1.1039x over previous
import jax
import jax.numpy as jnp
from jax import lax
from jax.experimental import pallas as pl
from jax.experimental.pallas import tpu as pltpu
from jax.experimental.pallas import tpu_sc as plsc

_NUM_LEVELS = 16
_TABLE_SIZE = 524288
_FEAT = 2
_N = 262144
_NC, _NS = 2, 16
_NW = _NC * _NS
_S = _N // _NW
_C = 512
_NCHUNK = _S // _C
_G = _C // 16
_IW = 512
_ROWS = (_C * _NUM_LEVELS * _FEAT) // _IW
_FU = 4

_P1 = -1640531535
_P2 = 805459861
_MASK = _TABLE_SIZE - 1


def _sc_body(pos_ref, scal_ref, table_ref, out_ref, posbuf, scalbuf, idxbuf0,
             idxbuf1, rowsbuf0, rowsbuf1, sem0, sem1):
    cid = lax.axis_index("c")
    sid = lax.axis_index("s")
    wid = sid * _NC + cid
    base = wid * _S
    pltpu.sync_copy(scal_ref, scalbuf)

    def compute(k, idxbuf):
        cb = base + k * _C
        pltpu.sync_copy(pos_ref.at[pl.ds(3 * cb, 3 * _C)], posbuf)

        def group_body(g, c2):
            iota = lax.iota(jnp.int32, 16)
            lane_col = iota * 32
            row = jnp.broadcast_to(g, (16,))
            off = jnp.broadcast_to(48 * g, (16,)) + 3 * iota
            xs = plsc.load_gather(posbuf, [off])
            ys = plsc.load_gather(posbuf, [off + 1])
            zs = plsc.load_gather(posbuf, [off + 2])
            for l in range(_NUM_LEVELS):
                s = scalbuf[l]
                sx = xs * s
                sy = ys * s
                sz = zs * s
                cx = sx.astype(jnp.int32)
                cy = sy.astype(jnp.int32)
                cz = sz.astype(jnp.int32)
                cx = cx + (sx > cx.astype(jnp.float32)).astype(jnp.int32)
                cy = cy + (sy > cy.astype(jnp.float32)).astype(jnp.int32)
                cz = cz + (sz > cz.astype(jnp.float32)).astype(jnp.int32)
                h = cx ^ (cy * jnp.int32(_P1)) ^ (cz * jnp.int32(_P2))
                h = (h & jnp.int32(_MASK)) + jnp.int32(l * _TABLE_SIZE)
                q0 = (lax.shift_right_logical(h, 7) * 256) + (h & 127)
                plsc.store_scatter(idxbuf, [row, lane_col + 2 * l], q0)
                plsc.store_scatter(idxbuf, [row, lane_col + 2 * l + 1],
                                   q0 + 128)
            return c2

        lax.fori_loop(0, _G, group_body, 0)

    def fire(idxbuf, rowsbuf, sem):
        def body(j, c2):
            for i in range(_FU):
                jj = j * _FU + i
                pltpu.make_async_copy(table_ref.at[idxbuf.at[jj]],
                                      rowsbuf.at[pl.ds(jj * _IW, _IW)],
                                      sem).start()
            return c2

        lax.fori_loop(0, _ROWS // _FU, body, 0)

    def drain(idxbuf, rowsbuf, sem):
        def body(j, c2):
            for i in range(_FU):
                jj = j * _FU + i
                pltpu.make_async_copy(table_ref.at[idxbuf.at[jj]],
                                      rowsbuf.at[pl.ds(jj * _IW, _IW)],
                                      sem).wait()
            return c2

        lax.fori_loop(0, _ROWS // _FU, body, 0)

    def writeback(k, rowsbuf):
        cb = base + k * _C
        pltpu.sync_copy(rowsbuf,
                        out_ref.at[pl.ds(cb * _NUM_LEVELS * _FEAT,
                                         _C * _NUM_LEVELS * _FEAT)])

    compute(0, idxbuf0)
    fire(idxbuf0, rowsbuf0, sem0)

    def pipe(k, carry):
        @pl.when(k % 2 == 1)
        def _():
            compute(k, idxbuf1)
            fire(idxbuf1, rowsbuf1, sem1)
            drain(idxbuf0, rowsbuf0, sem0)
            writeback(k - 1, rowsbuf0)

        @pl.when(k % 2 == 0)
        def _():
            compute(k, idxbuf0)
            fire(idxbuf0, rowsbuf0, sem0)
            drain(idxbuf1, rowsbuf1, sem1)
            writeback(k - 1, rowsbuf1)

        return carry

    lax.fori_loop(1, _NCHUNK, pipe, 0)
    drain(idxbuf1, rowsbuf1, sem1)
    writeback(_NCHUNK - 1, rowsbuf1)


def _sc_gather(pos_flat, scal_b, hash_table):
    mesh = plsc.VectorSubcoreMesh(core_axis_name="c", subcore_axis_name="s",
                                  num_cores=_NC, num_subcores=_NS)
    return pl.kernel(
        _sc_body,
        out_type=jax.ShapeDtypeStruct((_N * _NUM_LEVELS * _FEAT,), jnp.float32),
        mesh=mesh,
        scratch_types=[
            pltpu.VMEM((3 * _C,), jnp.float32),
            pltpu.VMEM((_NUM_LEVELS, 16), jnp.float32),
            pltpu.VMEM((_ROWS, _IW), jnp.int32),
            pltpu.VMEM((_ROWS, _IW), jnp.int32),
            pltpu.VMEM((_C * _NUM_LEVELS * _FEAT,), jnp.float32),
            pltpu.VMEM((_C * _NUM_LEVELS * _FEAT,), jnp.float32),
            pltpu.SemaphoreType.DMA,
            pltpu.SemaphoreType.DMA,
        ],
        compiler_params=pltpu.CompilerParams(use_tc_tiling_on_sc=False,
                                             needs_layout_passes=False),
    )(pos_flat, scal_b, hash_table)


_BM = 512
_M = _N // 4


def _mlp_body(enc_ref, dir_ref, w1, b1, w2, b2, w3a, w3b, b3, w4, b4, w5, b5,
              dens_ref, col_ref):
    x = enc_ref[...]
    h = jnp.maximum(jnp.dot(x, w1[...], preferred_element_type=jnp.float32)
                    + b1[...], 0.0)
    d128 = jnp.dot(h, w2[...], preferred_element_type=jnp.float32) + b2[...]
    dens_ref[...] = d128.T

    dd = dir_ref[...]
    rows = []
    for u in range(4):
        dx = dd[3 * u:3 * u + 1, :]
        dy = dd[3 * u + 1:3 * u + 2, :]
        dz = dd[3 * u + 2:3 * u + 3, :]
        xx = dx * dx
        yy = dy * dy
        zz = dz * dz
        rows += [
            0.28209479177387814 * jnp.ones_like(dx),
            0.4886025119029199 * dy,
            0.4886025119029199 * dz,
            0.4886025119029199 * dx,
            1.0925484305920792 * dx * dy,
            1.0925484305920792 * dy * dz,
            0.9461746957575601 * zz - 0.31539156525252,
            1.0925484305920792 * dx * dz,
            0.5462742152960396 * (xx - yy),
            0.5900435899266435 * dy * (3 * xx - yy),
            2.890611442640554 * dx * dy * dz,
            0.4570457994644658 * dy * (5 * zz - 1),
            0.3731763325901154 * dz * (5 * zz - 3),
            0.4570457994644658 * dx * (5 * zz - 1),
            1.445305721320277 * dz * (xx - yy),
            0.5900435899266435 * dx * (xx - 3 * yy),
        ]
    c4 = jnp.concatenate(rows, axis=0)
    shp = lax.dot_general(c4, w3b[...], (((0,), (0,)), ((), ())),
                          preferred_element_type=jnp.float32)
    densp = jnp.dot(d128, w3a[...], preferred_element_type=jnp.float32)
    xc = jnp.maximum(densp + shp + b3[...], 0.0)
    xc = jnp.maximum(jnp.dot(xc, w4[...], preferred_element_type=jnp.float32)
                     + b4[...], 0.0)
    col = jax.nn.sigmoid(
        jnp.dot(xc, w5[...], preferred_element_type=jnp.float32) + b5[...])
    col_ref[...] = col.T


def _mlp(enc128, dir4, W1, b1, W2, b2, W3, b3, W4, b4, W5, b5):
    from jax.scipy.linalg import block_diag

    def bd4(w):
        return block_diag(w, w, w, w)

    W1_4 = bd4(W1)
    W2_4 = bd4(W2)
    W3a4 = bd4(W3[:16, :])
    W3b4 = bd4(W3[16:, :])
    W4_4 = bd4(W4)
    W5_4 = bd4(W5)
    b1_4 = jnp.tile(b1, 4)[None, :]
    b2_4 = jnp.tile(b2, 4)[None, :]
    b3_4 = jnp.tile(b3, 4)[None, :]
    b4_4 = jnp.tile(b4, 4)[None, :]
    b5_4 = jnp.tile(b5, 4)[None, :]

    grid = (_M // _BM,)
    full = lambda a: pl.BlockSpec(a.shape, lambda i: (0, 0))
    dens_t, col_t = pl.pallas_call(
        _mlp_body,
        grid=grid,
        in_specs=[
            pl.BlockSpec((_BM, 128), lambda i: (i, 0)),
            pl.BlockSpec((12, _BM), lambda i: (0, i)),
            full(W1_4), full(b1_4),
            full(W2_4), full(b2_4),
            full(W3a4), full(W3b4), full(b3_4),
            full(W4_4), full(b4_4),
            full(W5_4), full(b5_4),
        ],
        out_specs=[
            pl.BlockSpec((64, _BM), lambda i: (0, i)),
            pl.BlockSpec((12, _BM), lambda i: (0, i)),
        ],
        out_shape=[
            jax.ShapeDtypeStruct((64, _M), jnp.float32),
            jax.ShapeDtypeStruct((12, _M), jnp.float32),
        ],
    )(enc128, dir4, W1_4, b1_4, W2_4, b2_4, W3a4, W3b4, b3_4, W4_4, b4_4,
      W5_4, b5_4)
    density = dens_t.reshape(4, 16, _M).transpose(1, 2, 0).reshape(16, _N).T
    color = col_t.reshape(4, 3, _M).transpose(1, 2, 0).reshape(3, _N).T
    return density, color


def kernel(position, direction, hash_table, W1, b1, W2, b2, W3, b3, W4, b4,
           W5, b5):
    levels = jnp.arange(_NUM_LEVELS)
    growth = jnp.exp((jnp.log(1024.0) - jnp.log(16.0)) / (_NUM_LEVELS - 1))
    scal = jnp.floor(16 * growth ** levels)
    scal_b = jnp.tile(scal.astype(jnp.float32)[:, None], (1, 16))
    pos_flat = position.reshape(-1)
    ht_flat = hash_table.reshape(_TABLE_SIZE * _NUM_LEVELS // 128, 128,
                                 _FEAT).transpose(0, 2, 1).reshape(-1)
    enc_rows = _sc_gather(pos_flat, scal_b, ht_flat)
    enc128 = enc_rows.reshape(_M, 128)
    dir4 = direction.reshape(_M, 4, 3).transpose(1, 2, 0).reshape(12, _M)
    density, color = _mlp(enc128, dir4, W1, b1, W2, b2, W3, b3, W4, b4,
                          W5, b5)
    return density, color


# speedup vs baseline: 78.5987x; 1.0416x over previous
import jax
import jax.numpy as jnp
from jax import lax
from jax.experimental import pallas as pl
from jax.experimental.pallas import tpu as pltpu
from jax.experimental.pallas import tpu_sc as plsc

_NUM_LEVELS = 16
_TABLE_SIZE = 524288
_FEAT = 2
_N = 262144
_NC, _NS = 2, 16
_NW = _NC * _NS
_S = _N // _NW
_C = 512
_NCHUNK = _S // _C
_G = _C // 16
_IW = 2048
_ROWS = (_C * _NUM_LEVELS * _FEAT) // _IW
_FU = 4

_P1 = -1640531535
_P2 = 805459861
_MASK = _TABLE_SIZE - 1


def _sc_body(pos_ref, scal_ref, table_ref, out_ref, posbuf, scalbuf, idxbuf0,
             idxbuf1, rowsbuf0, rowsbuf1, sem0, sem1):
    cid = lax.axis_index("c")
    sid = lax.axis_index("s")
    wid = sid * _NC + cid
    base = wid * _S
    pltpu.sync_copy(scal_ref, scalbuf)

    def compute(k, idxbuf):
        cb = base + k * _C
        pltpu.sync_copy(pos_ref.at[pl.ds(3 * cb, 3 * _C)], posbuf)

        def group_body(g, c2):
            iota = lax.iota(jnp.int32, 16)
            lane_col = jnp.broadcast_to((g & 3) * 512, (16,)) + iota * 32
            row = jnp.broadcast_to(lax.shift_right_logical(g, 2), (16,))
            off = jnp.broadcast_to(48 * g, (16,)) + 3 * iota
            xs = plsc.load_gather(posbuf, [off])
            ys = plsc.load_gather(posbuf, [off + 1])
            zs = plsc.load_gather(posbuf, [off + 2])
            for l in range(_NUM_LEVELS):
                s = scalbuf[l]
                sx = xs * s
                sy = ys * s
                sz = zs * s
                cx = sx.astype(jnp.int32)
                cy = sy.astype(jnp.int32)
                cz = sz.astype(jnp.int32)
                cx = cx + (sx > cx.astype(jnp.float32)).astype(jnp.int32)
                cy = cy + (sy > cy.astype(jnp.float32)).astype(jnp.int32)
                cz = cz + (sz > cz.astype(jnp.float32)).astype(jnp.int32)
                h = cx ^ (cy * jnp.int32(_P1)) ^ (cz * jnp.int32(_P2))
                h = (h & jnp.int32(_MASK)) + jnp.int32(l * _TABLE_SIZE)
                q0 = (lax.shift_right_logical(h, 7) * 256) + (h & 127)
                plsc.store_scatter(idxbuf, [row, lane_col + 2 * l], q0)
                plsc.store_scatter(idxbuf, [row, lane_col + 2 * l + 1],
                                   q0 + 128)
            return c2

        lax.fori_loop(0, _G, group_body, 0)

    def fire(idxbuf, rowsbuf, sem):
        def body(j, c2):
            for i in range(_FU):
                jj = j * _FU + i
                pltpu.make_async_copy(table_ref.at[idxbuf.at[jj]],
                                      rowsbuf.at[pl.ds(jj * _IW, _IW)],
                                      sem).start()
            return c2

        lax.fori_loop(0, _ROWS // _FU, body, 0)

    def drain(idxbuf, rowsbuf, sem):
        def body(j, c2):
            for i in range(_FU):
                jj = j * _FU + i
                pltpu.make_async_copy(table_ref.at[idxbuf.at[jj]],
                                      rowsbuf.at[pl.ds(jj * _IW, _IW)],
                                      sem).wait()
            return c2

        lax.fori_loop(0, _ROWS // _FU, body, 0)

    def writeback(k, rowsbuf):
        cb = base + k * _C
        pltpu.sync_copy(rowsbuf,
                        out_ref.at[pl.ds(cb * _NUM_LEVELS * _FEAT,
                                         _C * _NUM_LEVELS * _FEAT)])

    compute(0, idxbuf0)
    fire(idxbuf0, rowsbuf0, sem0)

    def pipe(k, carry):
        @pl.when(k % 2 == 1)
        def _():
            compute(k, idxbuf1)
            fire(idxbuf1, rowsbuf1, sem1)
            drain(idxbuf0, rowsbuf0, sem0)
            writeback(k - 1, rowsbuf0)

        @pl.when(k % 2 == 0)
        def _():
            compute(k, idxbuf0)
            fire(idxbuf0, rowsbuf0, sem0)
            drain(idxbuf1, rowsbuf1, sem1)
            writeback(k - 1, rowsbuf1)

        return carry

    lax.fori_loop(1, _NCHUNK, pipe, 0)
    drain(idxbuf1, rowsbuf1, sem1)
    writeback(_NCHUNK - 1, rowsbuf1)


def _sc_gather(pos_flat, scal_b, hash_table):
    mesh = plsc.VectorSubcoreMesh(core_axis_name="c", subcore_axis_name="s",
                                  num_cores=_NC, num_subcores=_NS)
    return pl.kernel(
        _sc_body,
        out_type=jax.ShapeDtypeStruct((_N * _NUM_LEVELS * _FEAT,), jnp.float32),
        mesh=mesh,
        scratch_types=[
            pltpu.VMEM((3 * _C,), jnp.float32),
            pltpu.VMEM((_NUM_LEVELS, 16), jnp.float32),
            pltpu.VMEM((_ROWS, _IW), jnp.int32),
            pltpu.VMEM((_ROWS, _IW), jnp.int32),
            pltpu.VMEM((_C * _NUM_LEVELS * _FEAT,), jnp.float32),
            pltpu.VMEM((_C * _NUM_LEVELS * _FEAT,), jnp.float32),
            pltpu.SemaphoreType.DMA,
            pltpu.SemaphoreType.DMA,
        ],
        compiler_params=pltpu.CompilerParams(use_tc_tiling_on_sc=False,
                                             needs_layout_passes=False),
    )(pos_flat, scal_b, hash_table)


_BM = 512
_M = _N // 4


def _mlp_body(enc_ref, dir_ref, w1, b1, w2, b2, w3a, w3b, b3, w4, b4, w5, b5,
              dens_ref, col_ref):
    x = enc_ref[...]
    h = jnp.maximum(jnp.dot(x, w1[...], preferred_element_type=jnp.float32)
                    + b1[...], 0.0)
    d128 = jnp.dot(h, w2[...], preferred_element_type=jnp.float32) + b2[...]
    dens_ref[...] = d128.T

    dd = dir_ref[...]
    rows = []
    for u in range(4):
        dx = dd[3 * u:3 * u + 1, :]
        dy = dd[3 * u + 1:3 * u + 2, :]
        dz = dd[3 * u + 2:3 * u + 3, :]
        xx = dx * dx
        yy = dy * dy
        zz = dz * dz
        rows += [
            0.28209479177387814 * jnp.ones_like(dx),
            0.4886025119029199 * dy,
            0.4886025119029199 * dz,
            0.4886025119029199 * dx,
            1.0925484305920792 * dx * dy,
            1.0925484305920792 * dy * dz,
            0.9461746957575601 * zz - 0.31539156525252,
            1.0925484305920792 * dx * dz,
            0.5462742152960396 * (xx - yy),
            0.5900435899266435 * dy * (3 * xx - yy),
            2.890611442640554 * dx * dy * dz,
            0.4570457994644658 * dy * (5 * zz - 1),
            0.3731763325901154 * dz * (5 * zz - 3),
            0.4570457994644658 * dx * (5 * zz - 1),
            1.445305721320277 * dz * (xx - yy),
            0.5900435899266435 * dx * (xx - 3 * yy),
        ]
    c4 = jnp.concatenate(rows, axis=0)
    shp = lax.dot_general(c4, w3b[...], (((0,), (0,)), ((), ())),
                          preferred_element_type=jnp.float32)
    densp = jnp.dot(d128, w3a[...], preferred_element_type=jnp.float32)
    xc = jnp.maximum(densp + shp + b3[...], 0.0)
    xc = jnp.maximum(jnp.dot(xc, w4[...], preferred_element_type=jnp.float32)
                     + b4[...], 0.0)
    col = jax.nn.sigmoid(
        jnp.dot(xc, w5[...], preferred_element_type=jnp.float32) + b5[...])
    col_ref[...] = col.T


def _mlp(enc128, dir4, W1, b1, W2, b2, W3, b3, W4, b4, W5, b5):
    from jax.scipy.linalg import block_diag

    def bd4(w):
        return block_diag(w, w, w, w)

    W1_4 = bd4(W1)
    W2_4 = bd4(W2)
    W3a4 = bd4(W3[:16, :])
    W3b4 = bd4(W3[16:, :])
    W4_4 = bd4(W4)
    W5_4 = bd4(W5)
    b1_4 = jnp.tile(b1, 4)[None, :]
    b2_4 = jnp.tile(b2, 4)[None, :]
    b3_4 = jnp.tile(b3, 4)[None, :]
    b4_4 = jnp.tile(b4, 4)[None, :]
    b5_4 = jnp.tile(b5, 4)[None, :]

    grid = (_M // _BM,)
    full = lambda a: pl.BlockSpec(a.shape, lambda i: (0, 0))
    dens_t, col_t = pl.pallas_call(
        _mlp_body,
        grid=grid,
        in_specs=[
            pl.BlockSpec((_BM, 128), lambda i: (i, 0)),
            pl.BlockSpec((12, _BM), lambda i: (0, i)),
            full(W1_4), full(b1_4),
            full(W2_4), full(b2_4),
            full(W3a4), full(W3b4), full(b3_4),
            full(W4_4), full(b4_4),
            full(W5_4), full(b5_4),
        ],
        out_specs=[
            pl.BlockSpec((64, _BM), lambda i: (0, i)),
            pl.BlockSpec((12, _BM), lambda i: (0, i)),
        ],
        out_shape=[
            jax.ShapeDtypeStruct((64, _M), jnp.float32),
            jax.ShapeDtypeStruct((12, _M), jnp.float32),
        ],
    )(enc128, dir4, W1_4, b1_4, W2_4, b2_4, W3a4, W3b4, b3_4, W4_4, b4_4,
      W5_4, b5_4)
    density = dens_t.reshape(4, 16, _M).transpose(1, 2, 0).reshape(16, _N).T
    color = col_t.reshape(4, 3, _M).transpose(1, 2, 0).reshape(3, _N).T
    return density, color


def kernel(position, direction, hash_table, W1, b1, W2, b2, W3, b3, W4, b4,
           W5, b5):
    levels = jnp.arange(_NUM_LEVELS)
    growth = jnp.exp((jnp.log(1024.0) - jnp.log(16.0)) / (_NUM_LEVELS - 1))
    scal = jnp.floor(16 * growth ** levels)
    scal_b = jnp.tile(scal.astype(jnp.float32)[:, None], (1, 16))
    pos_flat = position.reshape(-1)
    ht_flat = hash_table.reshape(_TABLE_SIZE * _NUM_LEVELS // 128, 128,
                                 _FEAT).transpose(0, 2, 1).reshape(-1)
    enc_rows = _sc_gather(pos_flat, scal_b, ht_flat)
    enc128 = enc_rows.reshape(_M, 128)
    dir4 = direction.reshape(_M, 4, 3).transpose(1, 2, 0).reshape(12, _M)
    density, color = _mlp(enc128, dir4, W1, b1, W2, b2, W3, b3, W4, b4,
                          W5, b5)
    return density, color
